# baseline (speedup 1.0000x reference)
import functools

import jax
import jax.numpy as jnp
from jax import lax
from jax.experimental import pallas as pl
from jax.experimental.pallas import tpu as pltpu
from jax.experimental.pallas import tpu_sc as plsc

_BATCH = 16384
_EMB = 32
_NTAB = 4
_ROWW = _NTAB * _EMB
_NC = 2
_NS = 16
_NW = _NC * _NS
_BPW = _BATCH // _NW
_CHUNK = 128
_NCHUNK = _BPW // _CHUNK

_mesh = plsc.VectorSubcoreMesh(core_axis_name="c", subcore_axis_name="s")


def _make_gather_kernel():
  @functools.partial(
      pl.kernel,
      mesh=_mesh,
      out_type=jax.ShapeDtypeStruct((_BATCH, _ROWW), jnp.float32),
      scratch_types=[
          pltpu.VMEM((_NTAB, _CHUNK), jnp.int32),
          pltpu.VMEM((_NTAB, _CHUNK, _ROWW), jnp.float32),
          pltpu.VMEM((_CHUNK, _ROWW), jnp.float32),
          pltpu.SemaphoreType.DMA,
      ],
  )
  def body(xT, wall, out_hbm, idx_v, rows_v, out_v, gsem):
    wid = lax.axis_index("s") * _NC + lax.axis_index("c")
    base = wid * _BPW
    for j in range(_NCHUNK):
      b0 = base + j * _CHUNK
      pltpu.sync_copy(xT.at[:, pl.ds(b0, _CHUNK)], idx_v)
      copies = [
          pltpu.async_copy(wall.at[idx_v.at[t]], rows_v.at[t], gsem)
          for t in range(_NTAB)
      ]
      for cp in copies:
        cp.wait()

      def select(b, _):
        for t in range(_NTAB):
          for k in range(_EMB // 16):
            c = t * _EMB + k * 16
            out_v[b, pl.ds(c, 16)] = rows_v[t, b, pl.ds(c, 16)]
        return ()

      lax.fori_loop(0, _CHUNK, select, ())
      pltpu.sync_copy(out_v, out_hbm.at[pl.ds(b0, _CHUNK)])

  return body


_gather = _make_gather_kernel()

_VOCAB = 100000
_TCB = 2048


def _tc_pack_body(wg_ref, wa_ref, wo_ref, wz_ref, out_ref):
  parts = [jnp.transpose(r[...])
           for r in (wg_ref, wa_ref, wo_ref, wz_ref)]
  out_ref[...] = jnp.concatenate(parts, axis=1)


_tc_pack = pl.pallas_call(
    _tc_pack_body,
    grid=(pl.cdiv(_VOCAB, _TCB),),
    in_specs=[pl.BlockSpec((_EMB, _TCB), lambda i: (0, i))] * _NTAB,
    out_specs=pl.BlockSpec((_TCB, _ROWW), lambda i: (i, 0)),
    out_shape=jax.ShapeDtypeStruct((_VOCAB, _ROWW), jnp.float32),
)


def kernel(x, W_gender, W_age, W_occupation, W_zip):
  w_all = _tc_pack(W_gender.T, W_age.T, W_occupation.T, W_zip.T)
  return _gather(x.T, w_all)


# speedup vs baseline: 1.2338x; 1.2338x over previous
import functools

import jax
import jax.numpy as jnp
from jax import lax
from jax.experimental import pallas as pl
from jax.experimental.pallas import tpu as pltpu
from jax.experimental.pallas import tpu_sc as plsc

_BATCH = 16384
_EMB = 32
_NTAB = 4
_ROWW = _NTAB * _EMB
_NC = 2
_NS = 16
_NW = _NC * _NS
_BPW = _BATCH // _NW
_CHUNK = 128
_NCHUNK = _BPW // _CHUNK

_mesh = plsc.VectorSubcoreMesh(core_axis_name="c", subcore_axis_name="s")


def _make_gather_kernel():
  @functools.partial(
      pl.kernel,
      mesh=_mesh,
      out_type=jax.ShapeDtypeStruct((_BATCH, _ROWW), jnp.float32),
      scratch_types=[
          pltpu.VMEM((_NTAB, _CHUNK), jnp.int32),
          pltpu.VMEM((_NTAB, _CHUNK, _ROWW), jnp.float32),
          pltpu.VMEM((_CHUNK, _ROWW), jnp.float32),
          pltpu.SemaphoreType.DMA,
      ],
  )
  def body(xT, wall, out_hbm, idx_v, rows_v, out_v, gsem):
    wid = lax.axis_index("s") * _NC + lax.axis_index("c")
    base = wid * _BPW
    for j in range(_NCHUNK):
      b0 = base + j * _CHUNK
      pltpu.sync_copy(xT.at[:, pl.ds(b0, _CHUNK)], idx_v)
      copies = [
          pltpu.async_copy(wall.at[idx_v.at[t]], rows_v.at[t], gsem)
          for t in range(_NTAB)
      ]
      for cp in copies:
        cp.wait()

      def select(b, _):
        for t in range(_NTAB):
          for k in range(_EMB // 16):
            c = t * _EMB + k * 16
            out_v[b, pl.ds(c, 16)] = rows_v[t, b, pl.ds(c, 16)]
        return ()

      lax.fori_loop(0, _CHUNK, select, ())
      pltpu.sync_copy(out_v, out_hbm.at[pl.ds(b0, _CHUNK)])

  return body


_gather = _make_gather_kernel()

_VOCAB = 100000
_TCB = 4096


def _tc_pack_body(wt_ref, out_ref):
  out_ref[...] = jnp.transpose(wt_ref[...])


_tc_pack = pl.pallas_call(
    _tc_pack_body,
    grid=(pl.cdiv(_VOCAB, _TCB),),
    in_specs=[pl.BlockSpec((_ROWW, _TCB), lambda i: (0, i))],
    out_specs=pl.BlockSpec((_TCB, _ROWW), lambda i: (i, 0)),
    out_shape=jax.ShapeDtypeStruct((_VOCAB, _ROWW), jnp.float32),
)


def kernel(x, W_gender, W_age, W_occupation, W_zip):
  wt_all = jnp.concatenate(
      (W_gender.T, W_age.T, W_occupation.T, W_zip.T), axis=0)
  w_all = _tc_pack(wt_all)
  return _gather(x.T, w_all)


# speedup vs baseline: 1.2950x; 1.0496x over previous
import functools

import jax
import jax.numpy as jnp
from jax import lax
from jax.experimental import pallas as pl
from jax.experimental.pallas import tpu as pltpu
from jax.experimental.pallas import tpu_sc as plsc

_BATCH = 16384
_EMB = 32
_NTAB = 4
_ROWW = _NTAB * _EMB
_NC = 2
_NS = 16
_NW = _NC * _NS
_BPW = _BATCH // _NW
_CHUNK = 128
_NCHUNK = _BPW // _CHUNK

_mesh = plsc.VectorSubcoreMesh(core_axis_name="c", subcore_axis_name="s")


def _make_gather_kernel():
  @functools.partial(
      pl.kernel,
      mesh=_mesh,
      out_type=jax.ShapeDtypeStruct((_BATCH, _ROWW), jnp.float32),
      scratch_types=[
          pltpu.VMEM((_NTAB, _CHUNK), jnp.int32),
          pltpu.VMEM((_NTAB, _CHUNK, _ROWW), jnp.float32),
          pltpu.VMEM((_CHUNK, _ROWW), jnp.float32),
          pltpu.SemaphoreType.DMA,
      ],
  )
  def body(xT, wall, out_hbm, idx_v, rows_v, out_v, gsem):
    wid = lax.axis_index("s") * _NC + lax.axis_index("c")
    base = wid * _BPW
    for j in range(_NCHUNK):
      b0 = base + j * _CHUNK
      pltpu.sync_copy(xT.at[:, pl.ds(b0, _CHUNK)], idx_v)
      copies = [
          pltpu.async_copy(wall.at[idx_v.at[t]], rows_v.at[t], gsem)
          for t in range(_NTAB)
      ]
      for cp in copies:
        cp.wait()

      def select(b, _):
        for t in range(_NTAB):
          for k in range(_EMB // 16):
            c = t * _EMB + k * 16
            out_v[b, pl.ds(c, 16)] = rows_v[t, b, pl.ds(c, 16)]
        return ()

      lax.fori_loop(0, _CHUNK, select, ())
      pltpu.sync_copy(out_v, out_hbm.at[pl.ds(b0, _CHUNK)])

  return body


_gather = _make_gather_kernel()

_VOCAB = 100000
_TCB = 8192


def _tc_pack_body(wt_ref, out_ref):
  out_ref[...] = jnp.transpose(wt_ref[...])


_tc_pack = pl.pallas_call(
    _tc_pack_body,
    grid=(pl.cdiv(_VOCAB, _TCB),),
    in_specs=[pl.BlockSpec((_ROWW, _TCB), lambda i: (0, i))],
    out_specs=pl.BlockSpec((_TCB, _ROWW), lambda i: (i, 0)),
    out_shape=jax.ShapeDtypeStruct((_VOCAB, _ROWW), jnp.float32),
)


def kernel(x, W_gender, W_age, W_occupation, W_zip):
  wt_all = jnp.concatenate(
      (W_gender.T, W_age.T, W_occupation.T, W_zip.T), axis=0)
  w_all = _tc_pack(wt_all)
  return _gather(x.T, w_all)
